# baseline (speedup 1.0000x reference)
import functools

import jax
import jax.numpy as jnp
from jax.experimental import pallas as pl
from jax.experimental.pallas import tpu as pltpu


def _knn_body(p8_ref, m8_ref, p16_ref, out_ref, minacc, *, nq, nk):
    i = pl.program_id(0)
    j = pl.program_id(1)
    p8 = p8_ref[...]
    d = p8.shape[1]

    bk = m8_ref.shape[0]
    quarter = jnp.full((1, d), 0.25, dtype=jnp.bfloat16)

    def half_dot(h):
        mh = m8_ref[pl.ds(h * (bk // 2), bk // 2), :]
        dot = jax.lax.dot_general(
            p8, mh, (((1,), (1,)), ((), ())),
            preferred_element_type=jnp.float32,
        )
        m16 = mh.astype(jnp.bfloat16)
        m2row = jax.lax.dot_general(
            quarter, m16 * m16, (((1,), (1,)), ((), ())),
            preferred_element_type=jnp.float32,
        )
        return dot, m2row

    def fold(dot, m2row):
        t = m2row[:, 0:128] + dot[:, 0:128]
        for c in range(1, dot.shape[1] // 128):
            sl = slice(c * 128, (c + 1) * 128)
            t = jnp.minimum(t, m2row[:, sl] + dot[:, sl])
        return t

    dot_a, m2_a = half_dot(0)
    dot_b, m2_b = half_dot(1)
    tmin = jnp.minimum(fold(dot_a, m2_a), fold(dot_b, m2_b))

    minacc[...] = jnp.where(j == 0, tmin, jnp.minimum(minacc[...], tmin))

    @pl.when(j == nk - 1)
    def _():
        pf = p16_ref[...].astype(jnp.float32)
        p2 = jnp.sum(pf * pf, axis=1, keepdims=True)
        rowmin = jnp.min(minacc[...], axis=1, keepdims=True)
        bmax = jnp.max(rowmin + p2)
        val = jnp.where(i == 0, bmax, jnp.maximum(out_ref[0, 0], bmax))
        out_ref[0, 0] = jnp.where(
            i == nq - 1, jnp.sqrt(jnp.maximum(val, 1e-12)), val
        )


def kernel(patches, memory_bank):
    q, d = patches.shape
    k, _ = memory_bank.shape
    bq = min(4096, q)
    bk = min(2048, k)
    nq, nk = q // bq, k // bk

    p8 = patches.astype(jnp.float8_e4m3fn)
    m8 = (memory_bank * -2.0).astype(jnp.float8_e4m3fn)
    p16 = patches.astype(jnp.bfloat16)

    out = pl.pallas_call(
        functools.partial(_knn_body, nq=nq, nk=nk),
        grid=(nq, nk),
        in_specs=[
            pl.BlockSpec((bq, d), lambda i, j: (i, 0)),
            pl.BlockSpec((bk, d), lambda i, j: (j, 0)),
            pl.BlockSpec((bq, d), lambda i, j: (i, 0)),
        ],
        out_specs=pl.BlockSpec(
            (1, 1), lambda i, j: (0, 0), memory_space=pltpu.SMEM
        ),
        out_shape=jax.ShapeDtypeStruct((1, 1), jnp.float32),
        scratch_shapes=[
            pltpu.VMEM((bq, 128), jnp.float32),
        ],
        compiler_params=pltpu.CompilerParams(
            dimension_semantics=("arbitrary", "arbitrary"),
        ),
    )(p8, m8, p16)
    return out[0, 0]


# speedup vs baseline: 1.0548x; 1.0548x over previous
import functools

import jax
import jax.numpy as jnp
from jax.experimental import pallas as pl
from jax.experimental.pallas import tpu as pltpu


def _knn_body(p8_ref, m8_ref, p16_ref, out_ref, minacc, *, nq, nk):
    i = pl.program_id(0)
    j = pl.program_id(1)
    p8 = p8_ref[...]
    m8 = m8_ref[...]
    d = p8.shape[1]

    dot = jax.lax.dot_general(
        p8, m8, (((1,), (1,)), ((), ())), preferred_element_type=jnp.float32
    )

    m16 = m8.astype(jnp.bfloat16)
    quarter = jnp.full((1, d), 0.25, dtype=jnp.bfloat16)
    m2row = jax.lax.dot_general(
        quarter, m16 * m16, (((1,), (1,)), ((), ())),
        preferred_element_type=jnp.float32,
    )

    bk = dot.shape[1]
    tmin = m2row[:, 0:128] + dot[:, 0:128]
    for c in range(1, bk // 128):
        sl = slice(c * 128, (c + 1) * 128)
        tmin = jnp.minimum(tmin, m2row[:, sl] + dot[:, sl])

    minacc[...] = jnp.where(j == 0, tmin, jnp.minimum(minacc[...], tmin))

    @pl.when(j == nk - 1)
    def _():
        pf = p16_ref[...].astype(jnp.float32)
        p2 = jnp.sum(pf * pf, axis=1, keepdims=True)
        rowmin = jnp.min(minacc[...], axis=1, keepdims=True)
        bmax = jnp.max(rowmin + p2)
        val = jnp.where(i == 0, bmax, jnp.maximum(out_ref[0, 0], bmax))
        out_ref[0, 0] = jnp.where(
            i == nq - 1, jnp.sqrt(jnp.maximum(val, 1e-12)), val
        )


def kernel(patches, memory_bank):
    q, d = patches.shape
    k, _ = memory_bank.shape
    bq = min(4096, q)
    bk = min(2048, k)
    nq, nk = q // bq, k // bk

    p8 = patches.astype(jnp.float8_e4m3fn)
    m8 = (memory_bank * -2.0).astype(jnp.float8_e4m3fn)
    p16 = patches.astype(jnp.bfloat16)

    out = pl.pallas_call(
        functools.partial(_knn_body, nq=nq, nk=nk),
        grid=(nq, nk),
        in_specs=[
            pl.BlockSpec((bq, d), lambda i, j: (i, 0)),
            pl.BlockSpec((bk, d), lambda i, j: (j, 0)),
            pl.BlockSpec((bq, d), lambda i, j: (i, 0)),
        ],
        out_specs=pl.BlockSpec(
            (1, 1), lambda i, j: (0, 0), memory_space=pltpu.SMEM
        ),
        out_shape=jax.ShapeDtypeStruct((1, 1), jnp.float32),
        scratch_shapes=[
            pltpu.VMEM((bq, 128), jnp.float32),
        ],
        compiler_params=pltpu.CompilerParams(
            dimension_semantics=("arbitrary", "arbitrary"),
        ),
    )(p8, m8, p16)
    return out[0, 0]
